# Initial kernel scaffold; baseline (speedup 1.0000x reference)
import jax
import jax.numpy as jnp
from jax.experimental import pallas as pl


def kernel(output, labels, images, reconstructions):
    raise NotImplementedError("write your pallas kernel here")


# speedup vs baseline: 1.1444x; 1.1444x over previous
import jax
import jax.numpy as jnp
from jax.experimental import pallas as pl
from jax.experimental.pallas import tpu as pltpu

_GAMMA = 2.0
_ALPHA = 0.5
_K = 16

_ROWS = 2592
_LANES = 640
_ROW_CHUNK = 216
_N_ROW_STEPS = _ROWS // _ROW_CHUNK

_REC_ROWS = 55296
_REC_CHUNK = 3456
_N_REC_STEPS = _REC_ROWS // _REC_CHUNK

_NEG_INF = float("-inf")


def _lane_iota(shape, dim):
    return jax.lax.broadcasted_iota(jnp.int32, shape, dim)


def _stable_softplus(x):
    return jnp.maximum(x, 0.0) + jnp.log1p(jnp.exp(-jnp.abs(x)))


def _row_body(o_ref, l_ref, acc_ref, top_ref, chunks_ref):
    step = pl.program_id(0)

    O = o_ref[...]
    L = l_ref[...]
    shape = O.shape
    comp = _lane_iota(shape, 1) % 5

    Lc = jnp.where(comp == 0, L, 0.0)
    pos0 = jnp.where(Lc > 0.5, 1.0, 0.0)
    neg0 = jnp.where(Lc < -0.5, 1.0, 0.0)

    pos_cnt = jnp.sum(pos0)
    neg_cnt = jnp.sum(neg0)
    pos_correct = jnp.sum(jnp.where(O >= 0.0, pos0, 0.0))

    p = jax.nn.sigmoid(O)
    fpos = 0.5 * (1.0 - p) ** _GAMMA * _stable_softplus(-O)
    fpos_sum = jnp.sum(fpos * pos0)

    pm = pos0
    pos_full = pos0
    for _ in range(4):
        pm = pltpu.roll(pm, 1, 1)
        pos_full = pos_full + pm

    d = O - L
    ad = jnp.abs(d)
    sl1 = jnp.where(ad < 1.0, 0.5 * d * d, ad - 0.5)
    sl1 = sl1 * pos_full
    sl_sums = [jnp.sum(jnp.where(comp == c, sl1, 0.0)) for c in (1, 2, 3, 4)]

    lane128 = _lane_iota((1, 128), 1)
    contrib = jnp.zeros((1, 128), jnp.float32)
    for q, v in enumerate(
        [pos_cnt, neg_cnt, pos_correct, fpos_sum] + sl_sums
    ):
        contrib = jnp.where(lane128 == q, v, contrib)

    @pl.when(step == 0)
    def _():
        acc_ref[...] = jnp.zeros_like(acc_ref)

    acc_ref[...] += contrib

    ns = jnp.where(neg0 > 0.0, O, _NEG_INF)
    fi = _lane_iota(shape, 0) * shape[1] + _lane_iota(shape, 1)
    tvec = jnp.full((1, 128), _NEG_INF, jnp.float32)
    for i in range(_K):
        m = jnp.max(ns)
        am = jnp.min(jnp.where(ns == m, fi, jnp.int32(2**30)))
        ns = jnp.where(fi == am, _NEG_INF, ns)
        tvec = jnp.where(lane128 == i, m, tvec)
    chunks_ref[pl.ds(step, 1), :] = tvec

    @pl.when(step == _N_ROW_STEPS - 1)
    def _():
        allc = chunks_ref[...]
        cfi = _lane_iota(allc.shape, 0) * 128 + _lane_iota(allc.shape, 1)
        out = jnp.full((1, 128), _NEG_INF, jnp.float32)
        acs = allc
        for i in range(_K):
            m = jnp.max(acs)
            am = jnp.min(jnp.where(acs == m, cfi, jnp.int32(2**30)))
            acs = jnp.where(cfi == am, _NEG_INF, acs)
            out = jnp.where(lane128 == i, m, out)
        top_ref[...] = out


def _recon_body(img_ref, rec_ref, acc_ref):
    step = pl.program_id(0)

    @pl.when(step == 0)
    def _():
        acc_ref[...] = jnp.zeros_like(acc_ref)

    d = rec_ref[...] - img_ref[...]
    acc_ref[...] += jnp.sum(d * d, axis=0, keepdims=True)


def _combine_body(acc_ref, top_ref, sse_ref, fout_ref, iout_ref):
    A = acc_ref[...]
    lane = _lane_iota((1, 128), 1)

    def pick(q):
        return jnp.sum(jnp.where(lane == q, A, 0.0))

    pos_cnt = pick(0)
    neg_cnt = pick(1)
    pos_correct = pick(2)
    fpos_sum = pick(3)
    sl_sums = [pick(4 + c) for c in range(4)]

    sse = jnp.sum(sse_ref[...])

    v = top_ref[...]
    validf = jnp.where(v != _NEG_INF, 1.0, 0.0)
    pneg = jax.nn.sigmoid(v)
    fneg = 0.5 * pneg * pneg * _stable_softplus(v)
    fneg_sum = jnp.sum(fneg * validf)
    neg_correct = jnp.sum(jnp.where(v < 0.0, validf, 0.0))

    neg_k = jnp.minimum(neg_cnt, jnp.float32(_K))
    classify = (fpos_sum + fneg_sum) / (pos_cnt + neg_k)
    rl = [
        jnp.where(pos_cnt > 0.0, s / jnp.maximum(pos_cnt, 1.0), 0.0)
        for s in sl_sums
    ]
    recon = jnp.float32(1e-06) * sse / jnp.float32(7077888.0)
    loss = classify + rl[0] + rl[1] + rl[2] + rl[3] + recon

    fvals = jnp.zeros((1, 128), jnp.float32)
    for q, val in enumerate([loss, classify, rl[0], rl[1], rl[2], rl[3], recon]):
        fvals = jnp.where(lane == q, val, fvals)
    fout_ref[...] = fvals

    ivals = jnp.zeros((1, 128), jnp.int32)
    for q, val in enumerate([pos_correct, pos_cnt, neg_correct, neg_k]):
        ivals = jnp.where(lane == q, val.astype(jnp.int32), ivals)
    iout_ref[...] = ivals


def kernel(output, labels, images, reconstructions):
    o = output.reshape(_ROWS, _LANES)
    l = labels.reshape(_ROWS, _LANES)

    acc, top16 = pl.pallas_call(
        _row_body,
        grid=(_N_ROW_STEPS,),
        in_specs=[
            pl.BlockSpec((_ROW_CHUNK, _LANES), lambda i: (i, 0)),
            pl.BlockSpec((_ROW_CHUNK, _LANES), lambda i: (i, 0)),
        ],
        out_specs=[
            pl.BlockSpec((1, 128), lambda i: (0, 0)),
            pl.BlockSpec((1, 128), lambda i: (0, 0)),
        ],
        out_shape=[
            jax.ShapeDtypeStruct((1, 128), jnp.float32),
            jax.ShapeDtypeStruct((1, 128), jnp.float32),
        ],
        scratch_shapes=[pltpu.VMEM((_N_ROW_STEPS, 128), jnp.float32)],
    )(o, l)

    img = images.reshape(_REC_ROWS, 128)
    rec = reconstructions.reshape(_REC_ROWS, 128)
    sse = pl.pallas_call(
        _recon_body,
        grid=(_N_REC_STEPS,),
        in_specs=[
            pl.BlockSpec((_REC_CHUNK, 128), lambda i: (i, 0)),
            pl.BlockSpec((_REC_CHUNK, 128), lambda i: (i, 0)),
        ],
        out_specs=pl.BlockSpec((1, 128), lambda i: (0, 0)),
        out_shape=jax.ShapeDtypeStruct((1, 128), jnp.float32),
    )(img, rec)

    fv, iv = pl.pallas_call(
        _combine_body,
        out_shape=[
            jax.ShapeDtypeStruct((1, 128), jnp.float32),
            jax.ShapeDtypeStruct((1, 128), jnp.int32),
        ],
    )(acc, top16, sse)

    return (
        fv[0, 0], fv[0, 1], fv[0, 2], fv[0, 3], fv[0, 4], fv[0, 5],
        iv[0, 0], iv[0, 1], iv[0, 2], iv[0, 3], fv[0, 6],
    )


# speedup vs baseline: 25.1072x; 21.9389x over previous
import jax
import jax.numpy as jnp
from jax.experimental import pallas as pl
from jax.experimental.pallas import tpu as pltpu

_GAMMA = 2.0
_K = 16

_N_ROW_STEPS = 8

_REC_SPLIT = 4
_N_REC_STEPS = 8 * _REC_SPLIT

_NEG_INF = float("-inf")


def _iota(shape, dim):
    return jax.lax.broadcasted_iota(jnp.int32, shape, dim)


def _stable_softplus(x):
    return jnp.maximum(x, 0.0) + jnp.log1p(jnp.exp(-jnp.abs(x)))


def _row_body(o_ref, l_ref, acc_ref, top_ref, chunks_ref):
    step = pl.program_id(0)

    O = o_ref[0]
    L = l_ref[0]
    cls_o = O[:, :, 0]
    cls_l = L[:, :, 0]

    pos = jnp.where(cls_l > 0.5, 1.0, 0.0)
    neg = jnp.where(cls_l < -0.5, 1.0, 0.0)

    pos_cnt = jnp.sum(pos)
    neg_cnt = jnp.sum(neg)
    pos_correct = jnp.sum(jnp.where(cls_o >= 0.0, pos, 0.0))

    p = jax.nn.sigmoid(cls_o)
    fpos = 0.5 * (1.0 - p) ** _GAMMA * _stable_softplus(-cls_o)
    fpos_sum = jnp.sum(fpos * pos)

    d = O[:, :, 1:5] - L[:, :, 1:5]
    ad = jnp.abs(d)
    sl1 = jnp.where(ad < 1.0, 0.5 * d * d, ad - 0.5)
    sl1 = sl1 * pos[:, :, None]
    sl_sums = [jnp.sum(sl1[:, :, c]) for c in range(4)]

    lane128 = _iota((1, 128), 1)
    contrib = jnp.zeros((1, 128), jnp.float32)
    for q, v in enumerate(
        [pos_cnt, neg_cnt, pos_correct, fpos_sum] + sl_sums
    ):
        contrib = jnp.where(lane128 == q, v, contrib)

    @pl.when(step == 0)
    def _():
        acc_ref[...] = jnp.zeros_like(acc_ref)

    acc_ref[...] += contrib

    ns = jnp.where(neg > 0.0, cls_o, _NEG_INF)
    s = ns.shape
    fi = ((_iota(s, 0) * s[1] + _iota(s, 1)) * s[2] + _iota(s, 2)) * s[3] \
        + _iota(s, 3)
    tvec = jnp.full((1, 128), _NEG_INF, jnp.float32)
    for i in range(_K):
        m = jnp.max(ns)
        am = jnp.min(jnp.where(ns == m, fi, jnp.int32(2**30)))
        ns = jnp.where(fi == am, _NEG_INF, ns)
        tvec = jnp.where(lane128 == i, m, tvec)
    chunks_ref[pl.ds(step, 1), :] = tvec

    @pl.when(step == _N_ROW_STEPS - 1)
    def _():
        allc = chunks_ref[...]
        cfi = _iota(allc.shape, 0) * 128 + _iota(allc.shape, 1)
        out = jnp.full((1, 128), _NEG_INF, jnp.float32)
        acs = allc
        for i in range(_K):
            m = jnp.max(acs)
            am = jnp.min(jnp.where(acs == m, cfi, jnp.int32(2**30)))
            acs = jnp.where(cfi == am, _NEG_INF, acs)
            out = jnp.where(lane128 == i, m, out)
        top_ref[...] = out


def _recon_body(img_ref, rec_ref, acc_ref):
    step = pl.program_id(0) * _REC_SPLIT + pl.program_id(1)

    @pl.when(step == 0)
    def _():
        acc_ref[...] = jnp.zeros_like(acc_ref)

    d = rec_ref[0, 0] - img_ref[0, 0]
    sse = jnp.sum(d * d)
    lane128 = _iota((1, 128), 1)
    acc_ref[...] += jnp.where(lane128 == 0, sse, 0.0)


def _combine_body(acc_ref, top_ref, sse_ref, fout_ref, iout_ref):
    A = acc_ref[...]
    lane = _iota((1, 128), 1)

    def pick(q):
        return jnp.sum(jnp.where(lane == q, A, 0.0))

    pos_cnt = pick(0)
    neg_cnt = pick(1)
    pos_correct = pick(2)
    fpos_sum = pick(3)
    sl_sums = [pick(4 + c) for c in range(4)]

    sse = jnp.sum(sse_ref[...])

    v = top_ref[...]
    validf = jnp.where(v != _NEG_INF, 1.0, 0.0)
    pneg = jax.nn.sigmoid(v)
    fneg = 0.5 * pneg * pneg * _stable_softplus(v)
    fneg_sum = jnp.sum(fneg * validf)
    neg_correct = jnp.sum(jnp.where(v < 0.0, validf, 0.0))

    neg_k = jnp.minimum(neg_cnt, jnp.float32(_K))
    classify = (fpos_sum + fneg_sum) / (pos_cnt + neg_k)
    rl = [
        jnp.where(pos_cnt > 0.0, s / jnp.maximum(pos_cnt, 1.0), 0.0)
        for s in sl_sums
    ]
    recon = jnp.float32(1e-06) * sse / jnp.float32(7077888.0)
    loss = classify + rl[0] + rl[1] + rl[2] + rl[3] + recon

    fvals = jnp.zeros((1, 128), jnp.float32)
    for q, val in enumerate([loss, classify, rl[0], rl[1], rl[2], rl[3], recon]):
        fvals = jnp.where(lane == q, val, fvals)
    fout_ref[...] = fvals

    ivals = jnp.zeros((1, 128), jnp.int32)
    for q, val in enumerate([pos_correct, pos_cnt, neg_correct, neg_k]):
        ivals = jnp.where(lane == q, val.astype(jnp.int32), ivals)
    iout_ref[...] = ivals


def kernel(output, labels, images, reconstructions):
    ot = jnp.transpose(output, (0, 1, 4, 5, 2, 3))
    lt = jnp.transpose(labels, (0, 1, 4, 5, 2, 3))

    acc, top16 = pl.pallas_call(
        _row_body,
        grid=(_N_ROW_STEPS,),
        in_specs=[
            pl.BlockSpec((1, 24, 3, 5, 24, 24), lambda i: (i, 0, 0, 0, 0, 0)),
            pl.BlockSpec((1, 24, 3, 5, 24, 24), lambda i: (i, 0, 0, 0, 0, 0)),
        ],
        out_specs=[
            pl.BlockSpec((1, 128), lambda i: (0, 0)),
            pl.BlockSpec((1, 128), lambda i: (0, 0)),
        ],
        out_shape=[
            jax.ShapeDtypeStruct((1, 128), jnp.float32),
            jax.ShapeDtypeStruct((1, 128), jnp.float32),
        ],
        scratch_shapes=[pltpu.VMEM((_N_ROW_STEPS, 128), jnp.float32)],
    )(ot, lt)

    sse = pl.pallas_call(
        _recon_body,
        grid=(8, _REC_SPLIT),
        in_specs=[
            pl.BlockSpec((1, 1, 96 // _REC_SPLIT, 96, 96),
                         lambda i, j: (i, 0, j, 0, 0)),
            pl.BlockSpec((1, 1, 96 // _REC_SPLIT, 96, 96),
                         lambda i, j: (i, 0, j, 0, 0)),
        ],
        out_specs=pl.BlockSpec((1, 128), lambda i, j: (0, 0)),
        out_shape=jax.ShapeDtypeStruct((1, 128), jnp.float32),
    )(images, reconstructions)

    fv, iv = pl.pallas_call(
        _combine_body,
        out_shape=[
            jax.ShapeDtypeStruct((1, 128), jnp.float32),
            jax.ShapeDtypeStruct((1, 128), jnp.int32),
        ],
    )(acc, top16, sse)

    return (
        fv[0, 0], fv[0, 1], fv[0, 2], fv[0, 3], fv[0, 4], fv[0, 5],
        iv[0, 0], iv[0, 1], iv[0, 2], iv[0, 3], fv[0, 6],
    )


# speedup vs baseline: 43.7626x; 1.7430x over previous
import jax
import jax.numpy as jnp
from jax.experimental import pallas as pl
from jax.experimental.pallas import tpu as pltpu

_GAMMA = 2.0
_K = 16

_N_ROW_STEPS = 8
_PLANES_PER_STEP = 72
_N_PLANES = _N_ROW_STEPS * _PLANES_PER_STEP

_REC_SPLIT = 4

_NEG_INF = float("-inf")


def _iota(shape, dim):
    return jax.lax.broadcasted_iota(jnp.int32, shape, dim)


def _stable_softplus(x):
    return jnp.maximum(x, 0.0) + jnp.log1p(jnp.exp(-jnp.abs(x)))


def _row_body(o_ref, l_ref, acc_ref, top_ref, ns_ref, pm_ref, cand_ref):
    step = pl.program_id(0)

    @pl.when(step == 0)
    def _():
        acc_ref[...] = jnp.zeros_like(acc_ref)
        pm_ref[...] = jnp.full_like(pm_ref, _NEG_INF)

    O = o_ref[0].reshape(_PLANES_PER_STEP, 5, 24, 24)
    L = l_ref[0].reshape(_PLANES_PER_STEP, 5, 24, 24)
    cls_o = O[:, 0]
    cls_l = L[:, 0]

    pos = jnp.where(cls_l > 0.5, 1.0, 0.0)
    neg = jnp.where(cls_l < -0.5, 1.0, 0.0)

    p = jax.nn.sigmoid(cls_o)
    fpos = 0.5 * (1.0 - p) ** _GAMMA * _stable_softplus(-cls_o)

    d = O[:, 1:5] - L[:, 1:5]
    ad = jnp.abs(d)
    sl1 = jnp.where(ad < 1.0, 0.5 * d * d, ad - 0.5)
    sl1 = sl1 * pos[:, None]

    parts = [
        jnp.sum(pos, axis=(0, 1)),
        jnp.sum(neg, axis=(0, 1)),
        jnp.sum(jnp.where(cls_o >= 0.0, pos, 0.0), axis=(0, 1)),
        jnp.sum(fpos * pos, axis=(0, 1)),
        jnp.sum(sl1[:, 0], axis=(0, 1)),
        jnp.sum(sl1[:, 1], axis=(0, 1)),
        jnp.sum(sl1[:, 2], axis=(0, 1)),
        jnp.sum(sl1[:, 3], axis=(0, 1)),
    ]
    for q, v in enumerate(parts):
        acc_ref[pl.ds(q, 1), pl.ds(0, 24)] += v.reshape(1, 24)

    ns = jnp.where(neg > 0.0, cls_o, _NEG_INF)
    ns_ref[pl.ds(step * _PLANES_PER_STEP, _PLANES_PER_STEP)] = ns
    pmax = jnp.max(ns, axis=(1, 2))
    pm_ref[pl.ds(step, 1), pl.ds(0, _PLANES_PER_STEP)] = pmax.reshape(1, -1)

    @pl.when(step == _N_ROW_STEPS - 1)
    def _():
        pm = pm_ref[...]
        fi = _iota(pm.shape, 0) * 128 + _iota(pm.shape, 1)
        for i in range(_K):
            m = jnp.max(pm)
            am = jnp.min(jnp.where(pm == m, fi, jnp.int32(2**30)))
            pm = jnp.where(fi == am, _NEG_INF, pm)
            chunk = am // 128
            plane = chunk * _PLANES_PER_STEP + (am - chunk * 128)
            cand_ref[pl.ds(i, 1)] = ns_ref[pl.ds(plane, 1)]

        cand = cand_ref[...]
        s = cand.shape
        cfi = (_iota(s, 0) * s[1] + _iota(s, 1)) * s[2] + _iota(s, 2)
        lane128 = _iota((1, 128), 1)
        out = jnp.full((1, 128), _NEG_INF, jnp.float32)
        for i in range(_K):
            m = jnp.max(cand)
            am = jnp.min(jnp.where(cand == m, cfi, jnp.int32(2**30)))
            cand = jnp.where(cfi == am, _NEG_INF, cand)
            out = jnp.where(lane128 == i, m, out)
        top_ref[...] = out


def _recon_body(img_ref, rec_ref, acc_ref):
    step = pl.program_id(0) * _REC_SPLIT + pl.program_id(1)

    @pl.when(step == 0)
    def _():
        acc_ref[...] = jnp.zeros_like(acc_ref)

    d = rec_ref[0, 0] - img_ref[0, 0]
    acc_ref[pl.ds(0, 1), pl.ds(0, 96)] += jnp.sum(d * d, axis=(0, 1)).reshape(1, 96)


def _combine_body(acc_ref, top_ref, sse_ref, fout_ref, iout_ref):
    A = acc_ref[...]
    lane = _iota((1, 128), 1)

    def pick(q):
        return jnp.sum(A[q])

    pos_cnt = pick(0)
    neg_cnt = pick(1)
    pos_correct = pick(2)
    fpos_sum = pick(3)
    sl_sums = [pick(4 + c) for c in range(4)]

    sse = jnp.sum(sse_ref[...])

    v = top_ref[...]
    validf = jnp.where(v != _NEG_INF, 1.0, 0.0)
    pneg = jax.nn.sigmoid(v)
    fneg = 0.5 * pneg * pneg * _stable_softplus(v)
    fneg_sum = jnp.sum(fneg * validf)
    neg_correct = jnp.sum(jnp.where(v < 0.0, validf, 0.0))

    neg_k = jnp.minimum(neg_cnt, jnp.float32(_K))
    classify = (fpos_sum + fneg_sum) / (pos_cnt + neg_k)
    rl = [
        jnp.where(pos_cnt > 0.0, s / jnp.maximum(pos_cnt, 1.0), 0.0)
        for s in sl_sums
    ]
    recon = jnp.float32(1e-06) * sse / jnp.float32(7077888.0)
    loss = classify + rl[0] + rl[1] + rl[2] + rl[3] + recon

    fvals = jnp.zeros((1, 128), jnp.float32)
    for q, val in enumerate([loss, classify, rl[0], rl[1], rl[2], rl[3], recon]):
        fvals = jnp.where(lane == q, val, fvals)
    fout_ref[...] = fvals

    ivals = jnp.zeros((1, 128), jnp.int32)
    for q, val in enumerate([pos_correct, pos_cnt, neg_correct, neg_k]):
        ivals = jnp.where(lane == q, val.astype(jnp.int32), ivals)
    iout_ref[...] = ivals


def kernel(output, labels, images, reconstructions):
    ot = jnp.transpose(output, (0, 1, 4, 5, 2, 3))
    lt = jnp.transpose(labels, (0, 1, 4, 5, 2, 3))

    acc, top16 = pl.pallas_call(
        _row_body,
        grid=(_N_ROW_STEPS,),
        in_specs=[
            pl.BlockSpec((1, 24, 3, 5, 24, 24), lambda i: (i, 0, 0, 0, 0, 0)),
            pl.BlockSpec((1, 24, 3, 5, 24, 24), lambda i: (i, 0, 0, 0, 0, 0)),
        ],
        out_specs=[
            pl.BlockSpec((8, 128), lambda i: (0, 0)),
            pl.BlockSpec((1, 128), lambda i: (0, 0)),
        ],
        out_shape=[
            jax.ShapeDtypeStruct((8, 128), jnp.float32),
            jax.ShapeDtypeStruct((1, 128), jnp.float32),
        ],
        scratch_shapes=[
            pltpu.VMEM((_N_PLANES, 24, 24), jnp.float32),
            pltpu.VMEM((_N_ROW_STEPS, 128), jnp.float32),
            pltpu.VMEM((_K, 24, 24), jnp.float32),
        ],
    )(ot, lt)

    sse = pl.pallas_call(
        _recon_body,
        grid=(8, _REC_SPLIT),
        in_specs=[
            pl.BlockSpec((1, 1, 96 // _REC_SPLIT, 96, 96),
                         lambda i, j: (i, 0, j, 0, 0)),
            pl.BlockSpec((1, 1, 96 // _REC_SPLIT, 96, 96),
                         lambda i, j: (i, 0, j, 0, 0)),
        ],
        out_specs=pl.BlockSpec((1, 128), lambda i, j: (0, 0)),
        out_shape=jax.ShapeDtypeStruct((1, 128), jnp.float32),
    )(images, reconstructions)

    fv, iv = pl.pallas_call(
        _combine_body,
        out_shape=[
            jax.ShapeDtypeStruct((1, 128), jnp.float32),
            jax.ShapeDtypeStruct((1, 128), jnp.int32),
        ],
    )(acc, top16, sse)

    return (
        fv[0, 0], fv[0, 1], fv[0, 2], fv[0, 3], fv[0, 4], fv[0, 5],
        iv[0, 0], iv[0, 1], iv[0, 2], iv[0, 3], fv[0, 6],
    )


# speedup vs baseline: 56.0547x; 1.2809x over previous
import jax
import jax.numpy as jnp
from jax.experimental import pallas as pl
from jax.experimental.pallas import tpu as pltpu

_GAMMA = 2.0
_K = 16

_N_ROW_STEPS = 8
_PLANES_PER_STEP = 72
_N_PLANES = _N_ROW_STEPS * _PLANES_PER_STEP

_REC_SPLIT = 4

_NEG_INF = float("-inf")


def _iota(shape, dim):
    return jax.lax.broadcasted_iota(jnp.int32, shape, dim)


def _stable_softplus(x):
    return jnp.maximum(x, 0.0) + jnp.log1p(jnp.exp(-jnp.abs(x)))


def _row_body(o_ref, l_ref, img_ref, rec_ref, acc_ref, top_ref, sse_ref,
              ns_ref, pm_ref, cand_ref):
    step = pl.program_id(0)

    @pl.when(step == 0)
    def _():
        acc_ref[...] = jnp.zeros_like(acc_ref)
        sse_ref[...] = jnp.zeros_like(sse_ref)
        pm_ref[...] = jnp.full_like(pm_ref, _NEG_INF)

    dr = rec_ref[0, 0] - img_ref[0, 0]
    sse_ref[pl.ds(0, 1), pl.ds(0, 96)] += (
        jnp.sum(dr * dr, axis=(0, 1)).reshape(1, 96))

    O = o_ref[0].reshape(_PLANES_PER_STEP, 5, 24, 24)
    L = l_ref[0].reshape(_PLANES_PER_STEP, 5, 24, 24)
    cls_o = O[:, 0]
    cls_l = L[:, 0]

    pos = jnp.where(cls_l > 0.5, 1.0, 0.0)
    neg = jnp.where(cls_l < -0.5, 1.0, 0.0)

    p = jax.nn.sigmoid(cls_o)
    fpos = 0.5 * (1.0 - p) ** _GAMMA * _stable_softplus(-cls_o)

    d = O[:, 1:5] - L[:, 1:5]
    ad = jnp.abs(d)
    sl1 = jnp.where(ad < 1.0, 0.5 * d * d, ad - 0.5)
    sl1 = sl1 * pos[:, None]

    parts = [
        jnp.sum(pos, axis=(0, 1)),
        jnp.sum(neg, axis=(0, 1)),
        jnp.sum(jnp.where(cls_o >= 0.0, pos, 0.0), axis=(0, 1)),
        jnp.sum(fpos * pos, axis=(0, 1)),
        jnp.sum(sl1[:, 0], axis=(0, 1)),
        jnp.sum(sl1[:, 1], axis=(0, 1)),
        jnp.sum(sl1[:, 2], axis=(0, 1)),
        jnp.sum(sl1[:, 3], axis=(0, 1)),
    ]
    for q, v in enumerate(parts):
        acc_ref[pl.ds(q, 1), pl.ds(0, 24)] += v.reshape(1, 24)

    ns = jnp.where(neg > 0.0, cls_o, _NEG_INF)
    ns_ref[pl.ds(step * _PLANES_PER_STEP, _PLANES_PER_STEP)] = ns
    pmax = jnp.max(ns, axis=(1, 2))
    pm_ref[pl.ds(step, 1), pl.ds(0, _PLANES_PER_STEP)] = pmax.reshape(1, -1)

    @pl.when(step == _N_ROW_STEPS - 1)
    def _():
        pm = pm_ref[...]
        fi = _iota(pm.shape, 0) * 128 + _iota(pm.shape, 1)
        for i in range(_K):
            m = jnp.max(pm)
            am = jnp.min(jnp.where(pm == m, fi, jnp.int32(2**30)))
            pm = jnp.where(fi == am, _NEG_INF, pm)
            chunk = am // 128
            plane = chunk * _PLANES_PER_STEP + (am - chunk * 128)
            cand_ref[pl.ds(i, 1)] = ns_ref[pl.ds(plane, 1)]

        cand = cand_ref[...]
        s = cand.shape
        cfi = (_iota(s, 0) * s[1] + _iota(s, 1)) * s[2] + _iota(s, 2)
        lane128 = _iota((1, 128), 1)
        out = jnp.full((1, 128), _NEG_INF, jnp.float32)
        for i in range(_K):
            m = jnp.max(cand)
            am = jnp.min(jnp.where(cand == m, cfi, jnp.int32(2**30)))
            cand = jnp.where(cfi == am, _NEG_INF, cand)
            out = jnp.where(lane128 == i, m, out)
        top_ref[...] = out


def _combine_body(acc_ref, top_ref, sse_ref, fout_ref, iout_ref):
    A = acc_ref[...]
    lane = _iota((1, 128), 1)

    def pick(q):
        return jnp.sum(A[q])

    pos_cnt = pick(0)
    neg_cnt = pick(1)
    pos_correct = pick(2)
    fpos_sum = pick(3)
    sl_sums = [pick(4 + c) for c in range(4)]

    sse = jnp.sum(sse_ref[...])

    v = top_ref[...]
    validf = jnp.where(v != _NEG_INF, 1.0, 0.0)
    pneg = jax.nn.sigmoid(v)
    fneg = 0.5 * pneg * pneg * _stable_softplus(v)
    fneg_sum = jnp.sum(fneg * validf)
    neg_correct = jnp.sum(jnp.where(v < 0.0, validf, 0.0))

    neg_k = jnp.minimum(neg_cnt, jnp.float32(_K))
    classify = (fpos_sum + fneg_sum) / (pos_cnt + neg_k)
    rl = [
        jnp.where(pos_cnt > 0.0, s / jnp.maximum(pos_cnt, 1.0), 0.0)
        for s in sl_sums
    ]
    recon = jnp.float32(1e-06) * sse / jnp.float32(7077888.0)
    loss = classify + rl[0] + rl[1] + rl[2] + rl[3] + recon

    fvals = jnp.zeros((1, 128), jnp.float32)
    for q, val in enumerate([loss, classify, rl[0], rl[1], rl[2], rl[3], recon]):
        fvals = jnp.where(lane == q, val, fvals)
    fout_ref[...] = fvals

    ivals = jnp.zeros((1, 128), jnp.int32)
    for q, val in enumerate([pos_correct, pos_cnt, neg_correct, neg_k]):
        ivals = jnp.where(lane == q, val.astype(jnp.int32), ivals)
    iout_ref[...] = ivals


def kernel(output, labels, images, reconstructions):
    ot = jnp.transpose(output, (0, 1, 4, 5, 2, 3))
    lt = jnp.transpose(labels, (0, 1, 4, 5, 2, 3))

    acc, top16, sse = pl.pallas_call(
        _row_body,
        grid=(_N_ROW_STEPS,),
        in_specs=[
            pl.BlockSpec((1, 24, 3, 5, 24, 24), lambda i: (i, 0, 0, 0, 0, 0)),
            pl.BlockSpec((1, 24, 3, 5, 24, 24), lambda i: (i, 0, 0, 0, 0, 0)),
            pl.BlockSpec((1, 1, 96, 96, 96), lambda i: (i, 0, 0, 0, 0)),
            pl.BlockSpec((1, 1, 96, 96, 96), lambda i: (i, 0, 0, 0, 0)),
        ],
        out_specs=[
            pl.BlockSpec((8, 128), lambda i: (0, 0)),
            pl.BlockSpec((1, 128), lambda i: (0, 0)),
            pl.BlockSpec((1, 128), lambda i: (0, 0)),
        ],
        out_shape=[
            jax.ShapeDtypeStruct((8, 128), jnp.float32),
            jax.ShapeDtypeStruct((1, 128), jnp.float32),
            jax.ShapeDtypeStruct((1, 128), jnp.float32),
        ],
        scratch_shapes=[
            pltpu.VMEM((_N_PLANES, 24, 24), jnp.float32),
            pltpu.VMEM((_N_ROW_STEPS, 128), jnp.float32),
            pltpu.VMEM((_K, 24, 24), jnp.float32),
        ],
    )(ot, lt, images, reconstructions)

    fv, iv = pl.pallas_call(
        _combine_body,
        out_shape=[
            jax.ShapeDtypeStruct((1, 128), jnp.float32),
            jax.ShapeDtypeStruct((1, 128), jnp.int32),
        ],
    )(acc, top16, sse)

    return (
        fv[0, 0], fv[0, 1], fv[0, 2], fv[0, 3], fv[0, 4], fv[0, 5],
        iv[0, 0], iv[0, 1], iv[0, 2], iv[0, 3], fv[0, 6],
    )


# speedup vs baseline: 63.7165x; 1.1367x over previous
import jax
import jax.numpy as jnp
from jax.experimental import pallas as pl
from jax.experimental.pallas import tpu as pltpu

_GAMMA = 2.0
_K = 16

_N_ROW_STEPS = 8
_PLANES_PER_STEP = 72
_N_PLANES = _N_ROW_STEPS * _PLANES_PER_STEP

_REC_SPLIT = 4

_NEG_INF = float("-inf")


def _iota(shape, dim):
    return jax.lax.broadcasted_iota(jnp.int32, shape, dim)


def _stable_softplus(x):
    return jnp.maximum(x, 0.0) + jnp.log1p(jnp.exp(-jnp.abs(x)))


def _row_body(o_ref, l_ref, img_ref, rec_ref, acc_ref, top_ref, sse_ref,
              ns_ref, pm_ref, cand_ref):
    step = pl.program_id(0)

    @pl.when(step == 0)
    def _():
        acc_ref[...] = jnp.zeros_like(acc_ref)
        sse_ref[...] = jnp.zeros_like(sse_ref)
        pm_ref[...] = jnp.full_like(pm_ref, _NEG_INF)

    dr = rec_ref[0, 0] - img_ref[0, 0]
    sse_ref[pl.ds(0, 1), pl.ds(0, 96)] += (
        jnp.sum(dr * dr, axis=(0, 1)).reshape(1, 96))

    O = o_ref[0].reshape(_PLANES_PER_STEP, 5, 24, 24)
    L = l_ref[0].reshape(_PLANES_PER_STEP, 5, 24, 24)
    cls_o = O[:, 0]
    cls_l = L[:, 0]

    def pack(x):
        g = x.reshape(_PLANES_PER_STEP // 4, 4, 24, 24)
        return jnp.concatenate([g[:, 0], g[:, 1], g[:, 2], g[:, 3]], axis=-1)

    cls_op = pack(cls_o)
    cls_lp = pack(cls_l)

    pos = jnp.where(cls_lp > 0.5, 1.0, 0.0)
    neg_p = jnp.where(cls_lp < -0.5, 1.0, 0.0)

    p = jax.nn.sigmoid(cls_op)
    fpos = 0.5 * (1.0 - p) ** _GAMMA * _stable_softplus(-cls_op)

    parts = [
        jnp.sum(pos, axis=(0, 1)),
        jnp.sum(neg_p, axis=(0, 1)),
        jnp.sum(jnp.where(cls_op >= 0.0, pos, 0.0), axis=(0, 1)),
        jnp.sum(fpos * pos, axis=(0, 1)),
    ]
    for c in range(4):
        d = pack(O[:, 1 + c]) - pack(L[:, 1 + c])
        ad = jnp.abs(d)
        sl1 = jnp.where(ad < 1.0, 0.5 * d * d, ad - 0.5)
        parts.append(jnp.sum(sl1 * pos, axis=(0, 1)))
    for q, v in enumerate(parts):
        acc_ref[pl.ds(q, 1), pl.ds(0, 96)] += v.reshape(1, 96)

    neg = jnp.where(cls_l < -0.5, 1.0, 0.0)
    ns = jnp.where(neg > 0.0, cls_o, _NEG_INF)
    ns_ref[pl.ds(step * _PLANES_PER_STEP, _PLANES_PER_STEP)] = ns
    pmax = jnp.max(ns, axis=(1, 2))
    pm_ref[pl.ds(step, 1), pl.ds(0, _PLANES_PER_STEP)] = pmax.reshape(1, -1)

    @pl.when(step == _N_ROW_STEPS - 1)
    def _():
        pm = pm_ref[...]
        fi = _iota(pm.shape, 0) * 128 + _iota(pm.shape, 1)
        for i in range(_K):
            m = jnp.max(pm)
            am = jnp.min(jnp.where(pm == m, fi, jnp.int32(2**30)))
            pm = jnp.where(fi == am, _NEG_INF, pm)
            chunk = am // 128
            plane = chunk * _PLANES_PER_STEP + (am - chunk * 128)
            cand_ref[pl.ds(i, 1)] = ns_ref[pl.ds(plane, 1)]

        cand = cand_ref[...]
        s = cand.shape
        cfi = (_iota(s, 0) * s[1] + _iota(s, 1)) * s[2] + _iota(s, 2)
        lane128 = _iota((1, 128), 1)
        out = jnp.full((1, 128), _NEG_INF, jnp.float32)
        for i in range(_K):
            m = jnp.max(cand)
            am = jnp.min(jnp.where(cand == m, cfi, jnp.int32(2**30)))
            cand = jnp.where(cfi == am, _NEG_INF, cand)
            out = jnp.where(lane128 == i, m, out)
        top_ref[...] = out


def _combine_body(acc_ref, top_ref, sse_ref, fout_ref, iout_ref):
    A = acc_ref[...]
    lane = _iota((1, 128), 1)

    def pick(q):
        return jnp.sum(A[q])

    pos_cnt = pick(0)
    neg_cnt = pick(1)
    pos_correct = pick(2)
    fpos_sum = pick(3)
    sl_sums = [pick(4 + c) for c in range(4)]

    sse = jnp.sum(sse_ref[...])

    v = top_ref[...]
    validf = jnp.where(v != _NEG_INF, 1.0, 0.0)
    pneg = jax.nn.sigmoid(v)
    fneg = 0.5 * pneg * pneg * _stable_softplus(v)
    fneg_sum = jnp.sum(fneg * validf)
    neg_correct = jnp.sum(jnp.where(v < 0.0, validf, 0.0))

    neg_k = jnp.minimum(neg_cnt, jnp.float32(_K))
    classify = (fpos_sum + fneg_sum) / (pos_cnt + neg_k)
    rl = [
        jnp.where(pos_cnt > 0.0, s / jnp.maximum(pos_cnt, 1.0), 0.0)
        for s in sl_sums
    ]
    recon = jnp.float32(1e-06) * sse / jnp.float32(7077888.0)
    loss = classify + rl[0] + rl[1] + rl[2] + rl[3] + recon

    fvals = jnp.zeros((1, 128), jnp.float32)
    for q, val in enumerate([loss, classify, rl[0], rl[1], rl[2], rl[3], recon]):
        fvals = jnp.where(lane == q, val, fvals)
    fout_ref[...] = fvals

    ivals = jnp.zeros((1, 128), jnp.int32)
    for q, val in enumerate([pos_correct, pos_cnt, neg_correct, neg_k]):
        ivals = jnp.where(lane == q, val.astype(jnp.int32), ivals)
    iout_ref[...] = ivals


def kernel(output, labels, images, reconstructions):
    ot = jnp.transpose(output, (0, 1, 4, 5, 2, 3))
    lt = jnp.transpose(labels, (0, 1, 4, 5, 2, 3))

    acc, top16, sse = pl.pallas_call(
        _row_body,
        grid=(_N_ROW_STEPS,),
        in_specs=[
            pl.BlockSpec((1, 24, 3, 5, 24, 24), lambda i: (i, 0, 0, 0, 0, 0)),
            pl.BlockSpec((1, 24, 3, 5, 24, 24), lambda i: (i, 0, 0, 0, 0, 0)),
            pl.BlockSpec((1, 1, 96, 96, 96), lambda i: (i, 0, 0, 0, 0)),
            pl.BlockSpec((1, 1, 96, 96, 96), lambda i: (i, 0, 0, 0, 0)),
        ],
        out_specs=[
            pl.BlockSpec((8, 128), lambda i: (0, 0)),
            pl.BlockSpec((1, 128), lambda i: (0, 0)),
            pl.BlockSpec((1, 128), lambda i: (0, 0)),
        ],
        out_shape=[
            jax.ShapeDtypeStruct((8, 128), jnp.float32),
            jax.ShapeDtypeStruct((1, 128), jnp.float32),
            jax.ShapeDtypeStruct((1, 128), jnp.float32),
        ],
        scratch_shapes=[
            pltpu.VMEM((_N_PLANES, 24, 24), jnp.float32),
            pltpu.VMEM((_N_ROW_STEPS, 128), jnp.float32),
            pltpu.VMEM((_K, 24, 24), jnp.float32),
        ],
    )(ot, lt, images, reconstructions)

    fv, iv = pl.pallas_call(
        _combine_body,
        out_shape=[
            jax.ShapeDtypeStruct((1, 128), jnp.float32),
            jax.ShapeDtypeStruct((1, 128), jnp.int32),
        ],
    )(acc, top16, sse)

    return (
        fv[0, 0], fv[0, 1], fv[0, 2], fv[0, 3], fv[0, 4], fv[0, 5],
        iv[0, 0], iv[0, 1], iv[0, 2], iv[0, 3], fv[0, 6],
    )


# speedup vs baseline: 63.7374x; 1.0003x over previous
import jax
import jax.numpy as jnp
from jax.experimental import pallas as pl
from jax.experimental.pallas import tpu as pltpu

_GAMMA = 2.0
_K = 16

_ROW_SPLIT = 3
_N_ROW_STEPS = 8 * _ROW_SPLIT
_PLANES_PER_STEP = 24
_N_PLANES = _N_ROW_STEPS * _PLANES_PER_STEP

_NEG_INF = float("-inf")


def _iota(shape, dim):
    return jax.lax.broadcasted_iota(jnp.int32, shape, dim)


def _stable_softplus(x):
    return jnp.maximum(x, 0.0) + jnp.log1p(jnp.exp(-jnp.abs(x)))


def _row_body(o_ref, l_ref, img_ref, rec_ref, acc_ref, top_ref, sse_ref,
              ns_ref, pm_ref, cand_ref):
    step = pl.program_id(0) * _ROW_SPLIT + pl.program_id(1)

    @pl.when(step == 0)
    def _():
        acc_ref[...] = jnp.zeros_like(acc_ref)
        sse_ref[...] = jnp.zeros_like(sse_ref)
        pm_ref[...] = jnp.full_like(pm_ref, _NEG_INF)

    dr = rec_ref[0, 0] - img_ref[0, 0]
    sse_ref[pl.ds(0, 1), pl.ds(0, 96)] += (
        jnp.sum(dr * dr, axis=(0, 1)).reshape(1, 96))

    O = o_ref[0].reshape(_PLANES_PER_STEP, 5, 24, 24)
    L = l_ref[0].reshape(_PLANES_PER_STEP, 5, 24, 24)
    cls_o = O[:, 0]
    cls_l = L[:, 0]

    def pack(x):
        g = x.reshape(_PLANES_PER_STEP // 4, 4, 24, 24)
        return jnp.concatenate([g[:, 0], g[:, 1], g[:, 2], g[:, 3]], axis=-1)

    cls_op = pack(cls_o)
    cls_lp = pack(cls_l)

    pos = jnp.where(cls_lp > 0.5, 1.0, 0.0)
    neg_p = jnp.where(cls_lp < -0.5, 1.0, 0.0)

    p = jax.nn.sigmoid(cls_op)
    fpos = 0.5 * (1.0 - p) ** _GAMMA * _stable_softplus(-cls_op)

    parts = [
        jnp.sum(pos, axis=(0, 1)),
        jnp.sum(neg_p, axis=(0, 1)),
        jnp.sum(jnp.where(cls_op >= 0.0, pos, 0.0), axis=(0, 1)),
        jnp.sum(fpos * pos, axis=(0, 1)),
    ]
    for c in range(4):
        d = pack(O[:, 1 + c]) - pack(L[:, 1 + c])
        ad = jnp.abs(d)
        sl1 = jnp.where(ad < 1.0, 0.5 * d * d, ad - 0.5)
        parts.append(jnp.sum(sl1 * pos, axis=(0, 1)))
    for q, v in enumerate(parts):
        acc_ref[pl.ds(q, 1), pl.ds(0, 96)] += v.reshape(1, 96)

    neg = jnp.where(cls_l < -0.5, 1.0, 0.0)
    ns = jnp.where(neg > 0.0, cls_o, _NEG_INF)
    ns_ref[pl.ds(step * _PLANES_PER_STEP, _PLANES_PER_STEP)] = ns
    pmax = jnp.max(ns, axis=(1, 2))
    pm_ref[pl.ds(step, 1), pl.ds(0, _PLANES_PER_STEP)] = pmax.reshape(1, -1)

    @pl.when(step == _N_ROW_STEPS - 1)
    def _():
        pm = pm_ref[...]
        fi = _iota(pm.shape, 0) * 128 + _iota(pm.shape, 1)
        for i in range(_K):
            m = jnp.max(pm)
            am = jnp.min(jnp.where(pm == m, fi, jnp.int32(2**30)))
            pm = jnp.where(fi == am, _NEG_INF, pm)
            chunk = am // 128
            plane = chunk * _PLANES_PER_STEP + (am - chunk * 128)
            cand_ref[pl.ds(i, 1)] = ns_ref[pl.ds(plane, 1)]

        cand = cand_ref[...]
        s = cand.shape
        cfi = (_iota(s, 0) * s[1] + _iota(s, 1)) * s[2] + _iota(s, 2)
        lane128 = _iota((1, 128), 1)
        out = jnp.full((1, 128), _NEG_INF, jnp.float32)
        for i in range(_K):
            m = jnp.max(cand)
            am = jnp.min(jnp.where(cand == m, cfi, jnp.int32(2**30)))
            cand = jnp.where(cfi == am, _NEG_INF, cand)
            out = jnp.where(lane128 == i, m, out)
        top_ref[...] = out


def _combine_body(acc_ref, top_ref, sse_ref, fout_ref, iout_ref):
    A = acc_ref[...]
    lane = _iota((1, 128), 1)

    def pick(q):
        return jnp.sum(A[q])

    pos_cnt = pick(0)
    neg_cnt = pick(1)
    pos_correct = pick(2)
    fpos_sum = pick(3)
    sl_sums = [pick(4 + c) for c in range(4)]

    sse = jnp.sum(sse_ref[...])

    v = top_ref[...]
    validf = jnp.where(v != _NEG_INF, 1.0, 0.0)
    pneg = jax.nn.sigmoid(v)
    fneg = 0.5 * pneg * pneg * _stable_softplus(v)
    fneg_sum = jnp.sum(fneg * validf)
    neg_correct = jnp.sum(jnp.where(v < 0.0, validf, 0.0))

    neg_k = jnp.minimum(neg_cnt, jnp.float32(_K))
    classify = (fpos_sum + fneg_sum) / (pos_cnt + neg_k)
    rl = [
        jnp.where(pos_cnt > 0.0, s / jnp.maximum(pos_cnt, 1.0), 0.0)
        for s in sl_sums
    ]
    recon = jnp.float32(1e-06) * sse / jnp.float32(7077888.0)
    loss = classify + rl[0] + rl[1] + rl[2] + rl[3] + recon

    fvals = jnp.zeros((1, 128), jnp.float32)
    for q, val in enumerate([loss, classify, rl[0], rl[1], rl[2], rl[3], recon]):
        fvals = jnp.where(lane == q, val, fvals)
    fout_ref[...] = fvals

    ivals = jnp.zeros((1, 128), jnp.int32)
    for q, val in enumerate([pos_correct, pos_cnt, neg_correct, neg_k]):
        ivals = jnp.where(lane == q, val.astype(jnp.int32), ivals)
    iout_ref[...] = ivals


def kernel(output, labels, images, reconstructions):
    ot = jnp.transpose(output, (0, 1, 4, 5, 2, 3))
    lt = jnp.transpose(labels, (0, 1, 4, 5, 2, 3))

    acc, top16, sse = pl.pallas_call(
        _row_body,
        grid=(8, _ROW_SPLIT),
        in_specs=[
            pl.BlockSpec((1, 8, 3, 5, 24, 24), lambda i, j: (i, j, 0, 0, 0, 0)),
            pl.BlockSpec((1, 8, 3, 5, 24, 24), lambda i, j: (i, j, 0, 0, 0, 0)),
            pl.BlockSpec((1, 1, 32, 96, 96), lambda i, j: (i, 0, j, 0, 0)),
            pl.BlockSpec((1, 1, 32, 96, 96), lambda i, j: (i, 0, j, 0, 0)),
        ],
        out_specs=[
            pl.BlockSpec((8, 128), lambda i, j: (0, 0)),
            pl.BlockSpec((1, 128), lambda i, j: (0, 0)),
            pl.BlockSpec((1, 128), lambda i, j: (0, 0)),
        ],
        out_shape=[
            jax.ShapeDtypeStruct((8, 128), jnp.float32),
            jax.ShapeDtypeStruct((1, 128), jnp.float32),
            jax.ShapeDtypeStruct((1, 128), jnp.float32),
        ],
        scratch_shapes=[
            pltpu.VMEM((_N_PLANES, 24, 24), jnp.float32),
            pltpu.VMEM((_N_ROW_STEPS, 128), jnp.float32),
            pltpu.VMEM((_K, 24, 24), jnp.float32),
        ],
    )(ot, lt, images, reconstructions)

    fv, iv = pl.pallas_call(
        _combine_body,
        out_shape=[
            jax.ShapeDtypeStruct((1, 128), jnp.float32),
            jax.ShapeDtypeStruct((1, 128), jnp.int32),
        ],
    )(acc, top16, sse)

    return (
        fv[0, 0], fv[0, 1], fv[0, 2], fv[0, 3], fv[0, 4], fv[0, 5],
        iv[0, 0], iv[0, 1], iv[0, 2], iv[0, 3], fv[0, 6],
    )
